# baseline (speedup 1.0000x reference)
import functools

import jax
import jax.numpy as jnp
from jax import lax
from jax.experimental import pallas as pl
from jax.experimental.pallas import tpu as pltpu
from jax.experimental.pallas import tpu_sc as plsc

B = 16384
L = 50
VOCAB = 2000
EMB = 32
MAX_NORM = 50.0
H1 = 256
OUT = 10

N_IDX = B * L


def _renorm_body(tab_ref, out_ref):
    t = tab_ref[...]
    sq = jnp.sum(t * t, axis=1, keepdims=True)
    norm = jnp.sqrt(sq)
    scale = jnp.where(norm > MAX_NORM, MAX_NORM / jnp.maximum(norm, 1e-12), 1.0)
    out_ref[...] = t * scale


def _renorm_table(emb_table):
    return pl.pallas_call(
        _renorm_body,
        out_shape=jax.ShapeDtypeStruct((VOCAB, EMB), jnp.float32),
    )(emb_table)


_NC = 2
_NS = 16
_NW = _NC * _NS
_PER_W = N_IDX // _NW
_CH = 1600
_NCHUNK = _PER_W // _CH


def _gather_body(per_w, n_chunks, tab_hbm, idx_hbm, out_hbm, idx0, idx1,
                 rows0, rows1, gs0, gs1, os0, os1):
    wid = lax.axis_index("s") * _NC + lax.axis_index("c")
    base_w = wid * per_w
    idx = (idx0, idx1)
    rows = (rows0, rows1)
    gsem = (gs0, gs1)
    osem = (os0, os1)
    gath = [None, None]
    outc = [None, None]
    pltpu.sync_copy(idx_hbm.at[pl.ds(base_w, _CH)], idx[0])
    gath[0] = pltpu.async_copy(tab_hbm.at[idx[0]], rows[0], gsem[0])
    for j in range(n_chunks):
        b = j & 1
        nb = 1 - b
        if j + 1 < n_chunks:
            pltpu.sync_copy(
                idx_hbm.at[pl.ds(base_w + (j + 1) * _CH, _CH)], idx[nb])
            if outc[nb] is not None:
                outc[nb].wait()
            gath[nb] = pltpu.async_copy(tab_hbm.at[idx[nb]], rows[nb], gsem[nb])
        gath[b].wait()
        outc[b] = pltpu.async_copy(
            rows[b],
            out_hbm.at[pl.ds(base_w + j * _CH, _CH)],
            osem[b])
    if n_chunks > 1:
        outc[(n_chunks - 2) & 1].wait()
    outc[(n_chunks - 1) & 1].wait()


def _sc_gather(table, idx_flat, n_idx):
    per_w = n_idx // _NW
    n_chunks = per_w // _CH
    mesh = plsc.VectorSubcoreMesh(core_axis_name="c", subcore_axis_name="s")
    return pl.kernel(
        functools.partial(_gather_body, per_w, n_chunks),
        out_type=jax.ShapeDtypeStruct((n_idx, EMB), jnp.float32),
        mesh=mesh,
        scratch_types=[
            pltpu.VMEM((_CH,), jnp.int32),
            pltpu.VMEM((_CH,), jnp.int32),
            pltpu.VMEM((_CH, EMB), jnp.float32),
            pltpu.VMEM((_CH, EMB), jnp.float32),
            pltpu.SemaphoreType.DMA,
            pltpu.SemaphoreType.DMA,
            pltpu.SemaphoreType.DMA,
            pltpu.SemaphoreType.DMA,
        ],
        compiler_params=pltpu.CompilerParams(use_tc_tiling_on_sc=False),
    )(table, idx_flat)


_TB = 512


def _mlp_body(nt, e_hbm, w1_ref, b1_ref, w2_ref, b2_ref, out_ref, buf, sems):
    i = pl.program_id(0)
    er = e_hbm.reshape(nt * _TB, L, EMB)

    def start(t, slot):
        pltpu.make_async_copy(
            er.at[pl.ds(t * _TB, _TB), :, :], buf.at[slot], sems.at[slot]
        ).start()

    @pl.when(i == 0)
    def _():
        start(0, 0)

    @pl.when(i + 1 < nt)
    def _():
        start(i + 1, (i + 1) % 2)

    slot = i % 2
    pltpu.make_async_copy(
        er.at[pl.ds(i * _TB, _TB), :, :], buf.at[slot], sems.at[slot]
    ).wait()
    e2 = buf[slot].reshape(_TB, L * EMB)
    h = jnp.dot(e2, w1_ref[...], preferred_element_type=jnp.float32)
    h = jnp.maximum(h + b1_ref[...], 0.0)
    logits = jnp.dot(h, w2_ref[...], preferred_element_type=jnp.float32)
    logits = logits + b2_ref[...]
    m = jnp.max(logits, axis=1, keepdims=True)
    ex = jnp.exp(logits - m)
    out_ref[...] = ex / jnp.sum(ex, axis=1, keepdims=True)


def _mlp(e, W1, b1, W2, b2):
    nb = (e.shape[0] * e.shape[1]) // (L * EMB)
    nt = nb // _TB
    return pl.pallas_call(
        functools.partial(_mlp_body, nt),
        grid=(nt,),
        in_specs=[
            pl.BlockSpec(memory_space=pl.ANY),
            pl.BlockSpec((L * EMB, H1), lambda i: (0, 0)),
            pl.BlockSpec((1, H1), lambda i: (0, 0)),
            pl.BlockSpec((H1, OUT), lambda i: (0, 0)),
            pl.BlockSpec((1, OUT), lambda i: (0, 0)),
        ],
        out_specs=pl.BlockSpec((_TB, OUT), lambda i: (i, 0)),
        out_shape=jax.ShapeDtypeStruct((nb, OUT), jnp.float32),
        scratch_shapes=[
            pltpu.VMEM((2, _TB, L, EMB), jnp.float32),
            pltpu.SemaphoreType.DMA((2,)),
        ],
    )(e, W1, b1, W2, b2)


_NSPLIT = 2


def kernel(x, emb_table, W1, b1, W2, b2):
    table = _renorm_table(emb_table.astype(jnp.float32))
    idx_flat = x.reshape(-1).astype(jnp.int32)
    b1r = b1.reshape(1, H1)
    b2r = b2.reshape(1, OUT)
    nh = N_IDX // _NSPLIT
    outs = []
    for k in range(_NSPLIT):
        e = _sc_gather(table, lax.slice(idx_flat, (k * nh,), ((k + 1) * nh,)),
                       nh)
        outs.append(_mlp(e, W1, b1r, W2, b2r))
    return jnp.concatenate(outs, axis=0)


# speedup vs baseline: 1.7264x; 1.7264x over previous
import functools

import jax
import jax.numpy as jnp
from jax import lax
from jax.experimental import pallas as pl
from jax.experimental.pallas import tpu as pltpu
from jax.experimental.pallas import tpu_sc as plsc

B = 16384
L = 50
VOCAB = 2000
EMB = 32
MAX_NORM = 50.0
H1 = 256
OUT = 10

N_IDX = B * L


def _renorm_body(tab_ref, out_ref):
    t = tab_ref[...]
    sq = jnp.sum(t * t, axis=1, keepdims=True)
    norm = jnp.sqrt(sq)
    scale = jnp.where(norm > MAX_NORM, MAX_NORM / jnp.maximum(norm, 1e-12), 1.0)
    out_ref[...] = t * scale


def _renorm_table(emb_table):
    return pl.pallas_call(
        _renorm_body,
        out_shape=jax.ShapeDtypeStruct((VOCAB, EMB), jnp.float32),
    )(emb_table)


_NC = 2
_NS = 16
_NW = _NC * _NS
_PER_W = N_IDX // _NW
_CH = 1600
_NCHUNK = _PER_W // _CH


def _gather_body(per_w, n_chunks, tab_hbm, idx_hbm, out_hbm, idx0, idx1,
                 rows0, rows1, gs0, gs1, os0, os1):
    wid = lax.axis_index("s") * _NC + lax.axis_index("c")
    base_w = wid * per_w
    idx = (idx0, idx1)
    rows = (rows0, rows1)
    gsem = (gs0, gs1)
    osem = (os0, os1)
    gath = [None, None]
    outc = [None, None]
    pltpu.sync_copy(idx_hbm.at[pl.ds(base_w, _CH)], idx[0])
    gath[0] = pltpu.async_copy(tab_hbm.at[idx[0]], rows[0], gsem[0])
    for j in range(n_chunks):
        b = j & 1
        nb = 1 - b
        if j + 1 < n_chunks:
            pltpu.sync_copy(
                idx_hbm.at[pl.ds(base_w + (j + 1) * _CH, _CH)], idx[nb])
            if outc[nb] is not None:
                outc[nb].wait()
            gath[nb] = pltpu.async_copy(tab_hbm.at[idx[nb]], rows[nb], gsem[nb])
        gath[b].wait()
        outc[b] = pltpu.async_copy(
            rows[b],
            out_hbm.at[pl.ds(base_w + j * _CH, _CH)],
            osem[b])
    if n_chunks > 1:
        outc[(n_chunks - 2) & 1].wait()
    outc[(n_chunks - 1) & 1].wait()


def _sc_gather(table, idx_flat, n_idx):
    per_w = n_idx // _NW
    n_chunks = per_w // _CH
    mesh = plsc.VectorSubcoreMesh(core_axis_name="c", subcore_axis_name="s")
    return pl.kernel(
        functools.partial(_gather_body, per_w, n_chunks),
        out_type=jax.ShapeDtypeStruct((n_idx, EMB), jnp.float32),
        mesh=mesh,
        scratch_types=[
            pltpu.VMEM((_CH,), jnp.int32),
            pltpu.VMEM((_CH,), jnp.int32),
            pltpu.VMEM((_CH, EMB), jnp.float32),
            pltpu.VMEM((_CH, EMB), jnp.float32),
            pltpu.SemaphoreType.DMA,
            pltpu.SemaphoreType.DMA,
            pltpu.SemaphoreType.DMA,
            pltpu.SemaphoreType.DMA,
        ],
        compiler_params=pltpu.CompilerParams(use_tc_tiling_on_sc=False),
    )(table, idx_flat)


_TB = 512


def _mlp_body(e_ref, w1_ref, b1_ref, w2_ref, b2_ref, out_ref):
    h = jnp.dot(e_ref[...], w1_ref[...], preferred_element_type=jnp.float32)
    h = jnp.maximum(h + b1_ref[...], 0.0)
    logits = jnp.dot(h, w2_ref[...], preferred_element_type=jnp.float32)
    logits = logits + b2_ref[...]
    m = jnp.max(logits, axis=1, keepdims=True)
    ex = jnp.exp(logits - m)
    out_ref[...] = ex / jnp.sum(ex, axis=1, keepdims=True)


def _mlp(e_flat, W1, b1, W2, b2):
    nb = e_flat.shape[0]
    grid = (nb // _TB,)
    return pl.pallas_call(
        _mlp_body,
        grid=grid,
        in_specs=[
            pl.BlockSpec((_TB, L * EMB), lambda i: (i, 0)),
            pl.BlockSpec((L * EMB, H1), lambda i: (0, 0)),
            pl.BlockSpec((1, H1), lambda i: (0, 0)),
            pl.BlockSpec((H1, OUT), lambda i: (0, 0)),
            pl.BlockSpec((1, OUT), lambda i: (0, 0)),
        ],
        out_specs=pl.BlockSpec((_TB, OUT), lambda i: (i, 0)),
        out_shape=jax.ShapeDtypeStruct((nb, OUT), jnp.float32),
    )(e_flat, W1, b1, W2, b2)


_NSPLIT = 4


def kernel(x, emb_table, W1, b1, W2, b2):
    table = _renorm_table(emb_table.astype(jnp.float32))
    idx_flat = x.reshape(-1).astype(jnp.int32)
    b1r = b1.reshape(1, H1)
    b2r = b2.reshape(1, OUT)
    nh = N_IDX // _NSPLIT
    outs = []
    for k in range(_NSPLIT):
        e = _sc_gather(table, lax.slice(idx_flat, (k * nh,), ((k + 1) * nh,)),
                       nh)
        e_flat = e.reshape(nh // L, L * EMB)
        outs.append(_mlp(e_flat, W1, b1r, W2, b2r))
    return jnp.concatenate(outs, axis=0)


# speedup vs baseline: 1.7940x; 1.0392x over previous
import functools

import jax
import jax.numpy as jnp
from jax import lax
from jax.experimental import pallas as pl
from jax.experimental.pallas import tpu as pltpu
from jax.experimental.pallas import tpu_sc as plsc

B = 16384
L = 50
VOCAB = 2000
EMB = 32
MAX_NORM = 50.0
H1 = 256
OUT = 10

N_IDX = B * L


def _renorm_body(tab_ref, out_ref):
    t = tab_ref[...]
    sq = jnp.sum(t * t, axis=1, keepdims=True)
    norm = jnp.sqrt(sq)
    scale = jnp.where(norm > MAX_NORM, MAX_NORM / jnp.maximum(norm, 1e-12), 1.0)
    out_ref[...] = t * scale


def _renorm_table(emb_table):
    return pl.pallas_call(
        _renorm_body,
        out_shape=jax.ShapeDtypeStruct((VOCAB, EMB), jnp.float32),
    )(emb_table)


_NC = 2
_NS = 16
_NW = _NC * _NS
_PER_W = N_IDX // _NW
_CH = 1600
_NCHUNK = _PER_W // _CH


def _gather_body(per_w, n_chunks, tab_hbm, idx_hbm, out_hbm, idx0, idx1,
                 rows0, rows1, gs0, gs1, os0, os1):
    wid = lax.axis_index("s") * _NC + lax.axis_index("c")
    base_w = wid * per_w
    idx = (idx0, idx1)
    rows = (rows0, rows1)
    gsem = (gs0, gs1)
    osem = (os0, os1)
    gath = [None, None]
    outc = [None, None]
    pltpu.sync_copy(idx_hbm.at[pl.ds(base_w, _CH)], idx[0])
    gath[0] = pltpu.async_copy(tab_hbm.at[idx[0]], rows[0], gsem[0])
    for j in range(n_chunks):
        b = j & 1
        nb = 1 - b
        if j + 1 < n_chunks:
            pltpu.sync_copy(
                idx_hbm.at[pl.ds(base_w + (j + 1) * _CH, _CH)], idx[nb])
            if outc[nb] is not None:
                outc[nb].wait()
            gath[nb] = pltpu.async_copy(tab_hbm.at[idx[nb]], rows[nb], gsem[nb])
        gath[b].wait()
        outc[b] = pltpu.async_copy(
            rows[b],
            out_hbm.at[pl.ds(base_w + j * _CH, _CH)],
            osem[b])
    if n_chunks > 1:
        outc[(n_chunks - 2) & 1].wait()
    outc[(n_chunks - 1) & 1].wait()


def _sc_gather(table, idx_flat, n_idx):
    per_w = n_idx // _NW
    n_chunks = per_w // _CH
    mesh = plsc.VectorSubcoreMesh(core_axis_name="c", subcore_axis_name="s")
    return pl.kernel(
        functools.partial(_gather_body, per_w, n_chunks),
        out_type=jax.ShapeDtypeStruct((n_idx, EMB), jnp.float32),
        mesh=mesh,
        scratch_types=[
            pltpu.VMEM((_CH,), jnp.int32),
            pltpu.VMEM((_CH,), jnp.int32),
            pltpu.VMEM((_CH, EMB), jnp.float32),
            pltpu.VMEM((_CH, EMB), jnp.float32),
            pltpu.SemaphoreType.DMA,
            pltpu.SemaphoreType.DMA,
            pltpu.SemaphoreType.DMA,
            pltpu.SemaphoreType.DMA,
        ],
        compiler_params=pltpu.CompilerParams(use_tc_tiling_on_sc=False),
    )(table, idx_flat)


_TB = 512


def _mlp_body(e_ref, w1_ref, b1_ref, w2_ref, b2_ref, out_ref):
    h = jnp.dot(e_ref[...], w1_ref[...], preferred_element_type=jnp.float32)
    h = jnp.maximum(h + b1_ref[...], 0.0)
    logits = jnp.dot(h, w2_ref[...], preferred_element_type=jnp.float32)
    logits = logits + b2_ref[...]
    m = jnp.max(logits, axis=1, keepdims=True)
    ex = jnp.exp(logits - m)
    out_ref[...] = ex / jnp.sum(ex, axis=1, keepdims=True)


def _mlp(e_flat, W1, b1, W2, b2):
    nb = e_flat.shape[0]
    grid = (nb // _TB,)
    return pl.pallas_call(
        _mlp_body,
        grid=grid,
        in_specs=[
            pl.BlockSpec((_TB, L * EMB), lambda i: (i, 0)),
            pl.BlockSpec((L * EMB, H1), lambda i: (0, 0)),
            pl.BlockSpec((1, H1), lambda i: (0, 0)),
            pl.BlockSpec((H1, OUT), lambda i: (0, 0)),
            pl.BlockSpec((1, OUT), lambda i: (0, 0)),
        ],
        out_specs=pl.BlockSpec((_TB, OUT), lambda i: (i, 0)),
        out_shape=jax.ShapeDtypeStruct((nb, OUT), jnp.float32),
    )(e_flat, W1, b1, W2, b2)


_SLICES = (2048, 4096, 5120, 5120)


def kernel(x, emb_table, W1, b1, W2, b2):
    table = _renorm_table(emb_table.astype(jnp.float32))
    idx_flat = x.reshape(-1).astype(jnp.int32)
    b1r = b1.reshape(1, H1)
    b2r = b2.reshape(1, OUT)
    outs = []
    off = 0
    for ns in _SLICES:
        nh = ns * L
        e = _sc_gather(table, lax.slice(idx_flat, (off,), (off + nh,)), nh)
        e_flat = e.reshape(ns, L * EMB)
        outs.append(_mlp(e_flat, W1, b1r, W2, b2r))
        off += nh
    return jnp.concatenate(outs, axis=0)
